# baseline (speedup 1.0000x reference)
import functools

import jax
import jax.numpy as jnp
from jax import lax
from jax.experimental import pallas as pl
from jax.experimental.pallas import tpu as pltpu
from jax.experimental.pallas import tpu_sc as plsc


def _make_sc_gather(batch, dim, num_workers, nc):
  b_per_w = batch // num_workers
  mesh = plsc.VectorSubcoreMesh(core_axis_name="c", subcore_axis_name="s")

  @functools.partial(
      pl.kernel,
      mesh=mesh,
      compiler_params=pltpu.CompilerParams(use_tc_tiling_on_sc=False),
      out_type=jax.ShapeDtypeStruct((batch, dim), jnp.float32),
      scratch_types=[
          pltpu.VMEM((b_per_w,), jnp.int32),
          pltpu.VMEM((b_per_w, dim), jnp.float32),
          pltpu.SemaphoreType.DMA,
      ],
  )
  def gather_kernel(idx_hbm, table_hbm, out_hbm, idx_v, rows_v, sem):
    wid = lax.axis_index("s") * nc + lax.axis_index("c")
    base = wid * b_per_w
    pltpu.sync_copy(idx_hbm.at[pl.ds(base, b_per_w)], idx_v)
    pltpu.async_copy(table_hbm.at[idx_v], rows_v, sem).wait()
    pltpu.sync_copy(rows_v, out_hbm.at[pl.ds(base, b_per_w)])

  return gather_kernel


def _mlp_body(x_ref, w1_ref, b1_ref, w2_ref, b2_ref, o_ref):
  h = jnp.dot(x_ref[...], w1_ref[...], preferred_element_type=jnp.float32)
  h = jnp.maximum(h + b1_ref[...], 0.0)
  o = jnp.dot(h, w2_ref[...], preferred_element_type=jnp.float32)
  o_ref[...] = o + b2_ref[...]


def _make_tc_mlp(batch, dim, hidden, blk):
  grid = batch // blk
  return pl.pallas_call(
      _mlp_body,
      grid=(grid,),
      in_specs=[
          pl.BlockSpec((blk, dim), lambda i: (i, 0)),
          pl.BlockSpec((dim, hidden), lambda i: (0, 0)),
          pl.BlockSpec((1, hidden), lambda i: (0, 0)),
          pl.BlockSpec((hidden, dim), lambda i: (0, 0)),
          pl.BlockSpec((1, dim), lambda i: (0, 0)),
      ],
      out_specs=pl.BlockSpec((blk, dim), lambda i: (i, 0)),
      out_shape=jax.ShapeDtypeStruct((batch, dim), jnp.float32),
  )


@jax.jit
def kernel(user_ids, emb_table, W1, b1, W2, b2):
  batch = user_ids.shape[0]
  num_users, dim = emb_table.shape
  hidden = W1.shape[1]

  info = plsc.get_sparse_core_info()
  nw = info.num_cores * info.num_subcores

  ids32 = user_ids.astype(jnp.int32)
  gathered = _make_sc_gather(batch, dim, nw, info.num_cores)(ids32, emb_table)

  mlp = _make_tc_mlp(batch, dim, hidden, blk=2048)
  return mlp(gathered, W1, b1.reshape(1, hidden), W2, b2.reshape(1, dim))


# speedup vs baseline: 1.7076x; 1.7076x over previous
import functools

import jax
import jax.numpy as jnp
from jax import lax
from jax.experimental import pallas as pl
from jax.experimental.pallas import tpu as pltpu
from jax.experimental.pallas import tpu_sc as plsc


def _make_sc_gather(batch, dim, num_workers, nc):
  b_per_w = batch // num_workers
  mesh = plsc.VectorSubcoreMesh(core_axis_name="c", subcore_axis_name="s")

  @functools.partial(
      pl.kernel,
      mesh=mesh,
      out_type=jax.ShapeDtypeStruct((batch, dim), jnp.float32),
      scratch_types=[
          pltpu.VMEM((b_per_w,), jnp.int32),
          pltpu.VMEM((b_per_w, dim), jnp.float32),
          pltpu.SemaphoreType.DMA,
      ],
  )
  def gather_kernel(idx_hbm, table_hbm, out_hbm, idx_v, rows_v, sem):
    wid = lax.axis_index("s") * nc + lax.axis_index("c")
    base = wid * b_per_w
    pltpu.sync_copy(idx_hbm.at[pl.ds(base, b_per_w)], idx_v)

    def fire(g, carry):
      vec = idx_v[pl.ds(g * 16, 16)]
      for j in range(16):
        row = vec[j]
        pltpu.async_copy(
            table_hbm.at[pl.ds(row, 1)],
            rows_v.at[pl.ds(g * 16 + j, 1)], sem)
      return carry

    lax.fori_loop(0, b_per_w // 16, fire, 0)
    pltpu.make_async_copy(
        table_hbm.at[pl.ds(0, b_per_w)], rows_v, sem).wait()
    pltpu.sync_copy(rows_v, out_hbm.at[pl.ds(base, b_per_w)])

  return gather_kernel


def _mlp_body(x_ref, w1_ref, b1_ref, w2_ref, b2_ref, o_ref):
  h = jnp.dot(x_ref[...], w1_ref[...], preferred_element_type=jnp.float32)
  h = jnp.maximum(h + b1_ref[...], 0.0)
  o = jnp.dot(h, w2_ref[...], preferred_element_type=jnp.float32)
  o_ref[...] = o + b2_ref[...]


def _make_tc_mlp(batch, dim, hidden, blk):
  grid = batch // blk
  return pl.pallas_call(
      _mlp_body,
      grid=(grid,),
      in_specs=[
          pl.BlockSpec((blk, dim), lambda i: (i, 0)),
          pl.BlockSpec((dim, hidden), lambda i: (0, 0)),
          pl.BlockSpec((1, hidden), lambda i: (0, 0)),
          pl.BlockSpec((hidden, dim), lambda i: (0, 0)),
          pl.BlockSpec((1, dim), lambda i: (0, 0)),
      ],
      out_specs=pl.BlockSpec((blk, dim), lambda i: (i, 0)),
      out_shape=jax.ShapeDtypeStruct((batch, dim), jnp.float32),
  )


@jax.jit
def kernel(user_ids, emb_table, W1, b1, W2, b2):
  batch = user_ids.shape[0]
  num_users, dim = emb_table.shape
  hidden = W1.shape[1]

  info = plsc.get_sparse_core_info()
  nw = info.num_cores * info.num_subcores

  ids32 = user_ids.astype(jnp.int32)
  gathered = _make_sc_gather(batch, dim, nw, info.num_cores)(ids32, emb_table)

  mlp = _make_tc_mlp(batch, dim, hidden, blk=2048)
  return mlp(gathered, W1, b1.reshape(1, hidden), W2, b2.reshape(1, dim))
